# baseline (speedup 1.0000x reference)
import jax
import jax.numpy as jnp
from jax import lax
from jax.experimental import pallas as pl
from jax.experimental.pallas import tpu as pltpu
from jax.experimental.pallas import tpu_sc as plsc

NUM_FEATURES = 26
VOCAB = 100000
EMBED_DIM = 32
BATCH = 16384

NUM_CORES = 2
NUM_SUBCORES = 16
NUM_WORKERS = NUM_CORES * NUM_SUBCORES

QCH = 4096
NQ = BATCH // QCH


def _sc_body(idx_hbm, tab_hbm, out_hbm, row_v, idx_v, ob0, ob1,
             sem_row, sem_idx, sem_wb0, sem_wb1):
    wid = lax.axis_index("s") * NUM_CORES + lax.axis_index("c")
    r = wid

    pltpu.async_copy(tab_hbm.at[0, r], row_v, sem_row)
    pltpu.async_copy(idx_hbm.at[0], idx_v, sem_idx)

    def gather_q(q, obuf):
        def jbody(j, carry):
            i16 = q * QCH + j * 16
            iv = idx_v[pl.ds(i16, 16)]
            obuf[pl.ds(j * 16, 16)] = plsc.load_gather(row_v, [iv])
            return carry

        lax.fori_loop(0, QCH // 16, jbody, 0, unroll=8)

    def feat_body(f, carry):
        pltpu.make_async_copy(tab_hbm.at[f, r], row_v, sem_row).wait()
        pltpu.make_async_copy(idx_hbm.at[f], idx_v, sem_idx).wait()

        gather_q(0, ob0)
        wb0a = pltpu.async_copy(ob0, out_hbm.at[f, r, pl.ds(0 * QCH, QCH)], sem_wb0)
        gather_q(1, ob1)
        wb1a = pltpu.async_copy(ob1, out_hbm.at[f, r, pl.ds(1 * QCH, QCH)], sem_wb1)
        wb0a.wait()
        gather_q(2, ob0)
        wb0b = pltpu.async_copy(ob0, out_hbm.at[f, r, pl.ds(2 * QCH, QCH)], sem_wb0)
        wb1a.wait()
        gather_q(3, ob1)
        wb1b = pltpu.async_copy(ob1, out_hbm.at[f, r, pl.ds(3 * QCH, QCH)], sem_wb1)

        @pl.when(f < NUM_FEATURES - 1)
        def _():
            pltpu.async_copy(tab_hbm.at[f + 1, r], row_v, sem_row)
            pltpu.async_copy(idx_hbm.at[f + 1], idx_v, sem_idx)

        wb0b.wait()
        wb1b.wait()
        return carry

    lax.fori_loop(0, NUM_FEATURES, feat_body, 0)


@jax.jit
def kernel(inputs, tables):
    tables_t = tables.transpose(0, 2, 1)
    inputs_t = inputs.T.astype(jnp.int32)
    run = pl.kernel(
        _sc_body,
        out_type=jax.ShapeDtypeStruct((NUM_FEATURES, EMBED_DIM, BATCH), jnp.float32),
        mesh=plsc.VectorSubcoreMesh(core_axis_name="c", subcore_axis_name="s"),
        compiler_params=pltpu.CompilerParams(needs_layout_passes=False),
        scratch_types=[
            pltpu.VMEM((VOCAB,), jnp.float32),
            pltpu.VMEM((BATCH,), jnp.int32),
            pltpu.VMEM((QCH,), jnp.float32),
            pltpu.VMEM((QCH,), jnp.float32),
            pltpu.SemaphoreType.DMA,
            pltpu.SemaphoreType.DMA,
            pltpu.SemaphoreType.DMA,
            pltpu.SemaphoreType.DMA,
        ],
    )
    out_t = run(inputs_t, tables_t)
    return out_t.transpose(0, 2, 1)
